# baseline (speedup 1.0000x reference)
import jax
import jax.numpy as jnp
from jax import lax
from jax.experimental import pallas as pl
from jax.experimental.pallas import tpu as pltpu
from jax.experimental.pallas import tpu_sc as plsc

VOCAB = 1_000_000
D = 32
NTOK = 819_200

_NC = 2
_NS = 16
_NW = _NC * _NS
_BPW = NTOK // _NW
_C = 128
_NCHUNK = _BPW // _C


def _emb_body(idx_hbm, table_hbm, out_hbm, idx_v, rows_v, gsem):
    wid = lax.axis_index("s") * _NC + lax.axis_index("c")
    base = wid * _BPW
    pltpu.sync_copy(idx_hbm.at[pl.ds(base, _BPW)], idx_v)

    def step(j, carry):
        ic = idx_v.at[pl.ds(j * _C, _C)]
        pltpu.async_copy(table_hbm.at[ic], rows_v, gsem).wait()
        pltpu.sync_copy(rows_v, out_hbm.at[pl.ds(base + j * _C, _C)])
        return carry

    lax.fori_loop(0, _NCHUNK, step, 0)


_emb = pl.kernel(
    _emb_body,
    out_type=jax.ShapeDtypeStruct((NTOK, D), jnp.float32),
    mesh=plsc.VectorSubcoreMesh(core_axis_name="c", subcore_axis_name="s"),
    scratch_types=[
        pltpu.VMEM((_BPW,), jnp.int32),
        pltpu.VMEM((_C, D), jnp.float32),
        pltpu.SemaphoreType.DMA,
    ],
    compiler_params=pltpu.CompilerParams(use_tc_tiling_on_sc=False),
)


@jax.jit
def kernel(x, weights):
    return _emb(x.astype(jnp.int32), weights)


# speedup vs baseline: 1.1482x; 1.1482x over previous
import jax
import jax.numpy as jnp
from jax import lax
from jax.experimental import pallas as pl
from jax.experimental.pallas import tpu as pltpu
from jax.experimental.pallas import tpu_sc as plsc

VOCAB = 1_000_000
D = 32
NTOK = 819_200

_NC = 2
_NS = 16
_NW = _NC * _NS
_BPW = NTOK // _NW
_C = 128
_K = 10
_KC = _K * _C
_NSUP = _BPW // _KC


def _emb_body(idx_hbm, table_hbm, out_hbm, idx_v, bufa, bufb, gsa, gsb, osa, osb):
    wid = lax.axis_index("s") * _NC + lax.axis_index("c")
    base = wid * _BPW
    pltpu.sync_copy(idx_hbm.at[pl.ds(base, _BPW)], idx_v)

    def fire(buf, sem, s):
        for k in range(_K):
            ic = idx_v.at[pl.ds(s * _KC + k * _C, _C)]
            pltpu.async_copy(table_hbm.at[ic], buf.at[pl.ds(k * _C, _C)], sem)

    def drain_g(buf, sem):
        pltpu.make_async_copy(table_hbm.at[pl.ds(0, _KC)], buf, sem).wait()

    def start_o(buf, sem, s):
        pltpu.async_copy(buf, out_hbm.at[pl.ds(base + s * _KC, _KC)], sem)

    def wait_o(buf, sem):
        pltpu.make_async_copy(buf, out_hbm.at[pl.ds(base, _KC)], sem).wait()

    fire(bufa, gsa, 0)

    def body(i2, carry):
        s = i2 * 2
        pl.when(s > 0)(lambda: wait_o(bufb, osb))
        fire(bufb, gsb, s + 1)
        drain_g(bufa, gsa)
        start_o(bufa, osa, s)
        def prefetch_a():
            wait_o(bufa, osa)
            fire(bufa, gsa, s + 2)
        pl.when(s + 2 < _NSUP)(prefetch_a)
        drain_g(bufb, gsb)
        start_o(bufb, osb, s + 1)
        return carry

    lax.fori_loop(0, _NSUP // 2, body, 0)
    wait_o(bufa, osa)
    wait_o(bufb, osb)


_emb = pl.kernel(
    _emb_body,
    out_type=jax.ShapeDtypeStruct((NTOK, D), jnp.float32),
    mesh=plsc.VectorSubcoreMesh(core_axis_name="c", subcore_axis_name="s"),
    scratch_types=[
        pltpu.VMEM((_BPW,), jnp.int32),
        pltpu.VMEM((_KC, D), jnp.float32),
        pltpu.VMEM((_KC, D), jnp.float32),
        pltpu.SemaphoreType.DMA,
        pltpu.SemaphoreType.DMA,
        pltpu.SemaphoreType.DMA,
        pltpu.SemaphoreType.DMA,
    ],
    compiler_params=pltpu.CompilerParams(use_tc_tiling_on_sc=False),
)


@jax.jit
def kernel(x, weights):
    return _emb(x.astype(jnp.int32), weights)
